# baseline (speedup 1.0000x reference)
import functools

import numpy as np
import jax
import jax.numpy as jnp
from jax import lax
from jax.experimental import pallas as pl
from jax.experimental.pallas import tpu as pltpu
from jax.experimental.pallas import tpu_sc as plsc

HIDDEN = 64
HPAD = 128
SEQ_LEN = 200
BATCH = 4096

NC = 2
NS = 16
NW = NC * NS
BTILE = BATCH // NW


def _pe_table() -> np.ndarray:
    seq_pos = np.arange(SEQ_LEN, dtype=np.float32) + 1.0
    power = np.arange(0, HIDDEN, 2, dtype=np.float32) / HIDDEN
    divisor = 10000.0 ** power
    ang = seq_pos[:, None] / divisor[None, :]
    pe = np.stack((np.sin(ang), np.cos(ang)), axis=-1)
    pe = pe.reshape(SEQ_LEN, HIDDEN)
    return np.ascontiguousarray(pe.reshape(SEQ_LEN // 2, 2 * HIDDEN))


_PE = _pe_table()


def _sc_body(idx_hbm, table_hbm, pe_hbm, out_hbm,
             idx_v, rows_v, trans_v, pe_v, gsem0, gsem1, osem0, osem1):
    wid = lax.axis_index("s") * NC + lax.axis_index("c")
    col0 = wid * BTILE
    gsem = (gsem0, gsem1)
    osem = (osem0, osem1)
    pltpu.sync_copy(pe_hbm, pe_v)
    pltpu.sync_copy(
        idx_hbm.at[pl.ds(0, SEQ_LEN), pl.ds(col0, BTILE)], idx_v)

    def gdesc(l, b):
        return pltpu.make_async_copy(
            table_hbm.at[idx_v.at[l]], rows_v.at[b], gsem[b])

    def odesc(l, b):
        return pltpu.make_async_copy(
            trans_v.at[b],
            out_hbm.at[l, pl.ds(0, HIDDEN), pl.ds(col0, BTILE)],
            osem[b])

    IOTA = lax.iota(jnp.int32, 16)

    gdesc(0, 0).start()

    @pl.loop(0, SEQ_LEN, step=2)
    def _pos2(l0):
        for u in range(2):
            l = l0 + u
            b = u
            if u == 0:
                gdesc(l + 1, 1 - b).start()

                @pl.when(l >= 2)
                def _():
                    odesc(l - 2, b).wait()
            else:
                @pl.when(l + 1 < SEQ_LEN)
                def _():
                    gdesc(l + 1, 1 - b).start()

                @pl.when(l >= 2)
                def _():
                    odesc(l - 2, b).wait()

            gdesc(l, b).wait()

            per = jnp.full((16,), l // 2, jnp.int32)
            p0v = jnp.full((16,), (l % 2) * HIDDEN, jnp.int32)

            @pl.loop(0, (BTILE // 16) * (HIDDEN // 16))
            def _blk(blk):
                g = blk // (HIDDEN // 16)
                h0m = blk % (HIDDEN // 16)
                rowv = jnp.full((16,), g * 16, jnp.int32) + IOTA
                colb = jnp.full((16,), h0m * 16, jnp.int32)
                for k in range(16):
                    rot = (IOTA + k) & 15
                    cr = colb + rot
                    v = plsc.load_gather(rows_v.at[b], [rowv, cr])
                    pe = plsc.load_gather(pe_v, [per, p0v + cr])
                    plsc.store_scatter(trans_v.at[b], [cr, rowv], v + pe)

            odesc(l, b).start()

    odesc(SEQ_LEN - 2, 0).wait()
    odesc(SEQ_LEN - 1, 1).wait()


@jax.jit
def _sc_embed(idx, table, pe):
    mesh = plsc.VectorSubcoreMesh(
        core_axis_name="c", subcore_axis_name="s", num_cores=NC, num_subcores=NS)
    fn = functools.partial(
        pl.kernel,
        out_type=jax.ShapeDtypeStruct((SEQ_LEN, HIDDEN, BATCH), jnp.float32),
        mesh=mesh,
        scratch_types=[
            pltpu.VMEM((SEQ_LEN, BTILE), jnp.int32),
            pltpu.VMEM((2, BTILE, HPAD), jnp.float32),
            pltpu.VMEM((2, HIDDEN, BTILE), jnp.float32),
            pltpu.VMEM((SEQ_LEN // 2, 2 * HIDDEN), jnp.float32),
            pltpu.SemaphoreType.DMA,
            pltpu.SemaphoreType.DMA,
            pltpu.SemaphoreType.DMA,
            pltpu.SemaphoreType.DMA,
        ],
        compiler_params=pltpu.CompilerParams(
            use_tc_tiling_on_sc=True, needs_layout_passes=False),
    )(_sc_body)
    return fn(idx, table, pe)


def kernel(inputs, table):
    idx = inputs.T.astype(jnp.int32)
    table = jnp.pad(table.astype(jnp.float32), ((0, 0), (0, HPAD - HIDDEN)))
    pe = jnp.asarray(_PE, dtype=jnp.float32)
    out = _sc_embed(idx, table, pe)
    return jnp.transpose(out, (2, 0, 1))


# speedup vs baseline: 1.7952x; 1.7952x over previous
import functools

import numpy as np
import jax
import jax.numpy as jnp
from jax import lax
from jax.experimental import pallas as pl
from jax.experimental.pallas import tpu as pltpu
from jax.experimental.pallas import tpu_sc as plsc

HIDDEN = 64
HPAD = 128
SEQ_LEN = 200
BATCH = 4096

NC = 2
NS = 16
NW = NC * NS
BTILE = BATCH // NW


def _pe_table() -> np.ndarray:
    seq_pos = np.arange(SEQ_LEN, dtype=np.float32) + 1.0
    power = np.arange(0, HIDDEN, 2, dtype=np.float32) / HIDDEN
    divisor = 10000.0 ** power
    ang = seq_pos[:, None] / divisor[None, :]
    pe = np.stack((np.sin(ang), np.cos(ang)), axis=-1)
    pe = pe.reshape(SEQ_LEN, HIDDEN)
    return np.ascontiguousarray(pe.reshape(SEQ_LEN // 2, 2 * HIDDEN))


_PE = _pe_table()


def _sc_body(idx_hbm, table_hbm, pe_hbm, out_hbm,
             idx_v, rows_v, trans_v, pe_v, gsem0, gsem1, osem0, osem1):
    wid = lax.axis_index("s") * NC + lax.axis_index("c")
    col0 = wid * BTILE
    gsem = (gsem0, gsem1)
    osem = (osem0, osem1)
    pltpu.sync_copy(pe_hbm, pe_v)
    pltpu.sync_copy(
        idx_hbm.at[pl.ds(0, SEQ_LEN), pl.ds(col0, BTILE)], idx_v)

    def gdesc(l, b):
        return pltpu.make_async_copy(
            table_hbm.at[idx_v.at[l]], rows_v.at[b], gsem[b])

    def odesc(l, b):
        return pltpu.make_async_copy(
            trans_v.at[b],
            out_hbm.at[l, pl.ds(0, HIDDEN), pl.ds(col0, BTILE)],
            osem[b])

    IOTA = lax.iota(jnp.int32, 16)

    gdesc(0, 0).start()

    @pl.loop(0, SEQ_LEN, step=2)
    def _pos2(l0):
        for u in range(2):
            l = l0 + u
            b = u
            if u == 0:
                gdesc(l + 1, 1 - b).start()

                @pl.when(l >= 2)
                def _():
                    odesc(l - 2, b).wait()
            else:
                @pl.when(l + 1 < SEQ_LEN)
                def _():
                    gdesc(l + 1, 1 - b).start()

                @pl.when(l >= 2)
                def _():
                    odesc(l - 2, b).wait()

            gdesc(l, b).wait()

            p0 = (l % 2) * HIDDEN
            pe_vecs = [pe_v[l // 2, pl.ds(p0 + 16 * h0, 16)]
                       for h0 in range(HIDDEN // 16)]

            @pl.loop(0, BTILE, step=8)
            def _pe(bb0):
                for k in range(8):
                    for h0 in range(HIDDEN // 16):
                        plsc.addupdate(
                            rows_v.at[b, bb0 + k, pl.ds(16 * h0, 16)],
                            pe_vecs[h0])

            @pl.loop(0, (BTILE // 16) * (HIDDEN // 16))
            def _blk(blk):
                g = blk // (HIDDEN // 16)
                h0m = blk % (HIDDEN // 16)
                rowv = jnp.full((16,), g * 16, jnp.int32) + IOTA
                colb = jnp.full((16,), h0m * 16, jnp.int32)
                for k0 in range(0, 16, 4):
                    crs, vals = [], []
                    for k in range(k0, k0 + 4):
                        rot = (IOTA + k) & 15
                        crs.append(colb + rot)
                        vals.append(
                            plsc.load_gather(rows_v.at[b], [rowv, crs[-1]]))
                    for cr, v in zip(crs, vals):
                        plsc.store_scatter(trans_v.at[b], [cr, rowv], v)

            odesc(l, b).start()

    odesc(SEQ_LEN - 2, 0).wait()
    odesc(SEQ_LEN - 1, 1).wait()


@jax.jit
def _sc_embed(idx, table, pe):
    mesh = plsc.VectorSubcoreMesh(
        core_axis_name="c", subcore_axis_name="s", num_cores=NC, num_subcores=NS)
    fn = functools.partial(
        pl.kernel,
        out_type=jax.ShapeDtypeStruct((SEQ_LEN, HIDDEN, BATCH), jnp.float32),
        mesh=mesh,
        scratch_types=[
            pltpu.VMEM((SEQ_LEN, BTILE), jnp.int32),
            pltpu.VMEM((2, BTILE, HPAD), jnp.float32),
            pltpu.VMEM((2, HIDDEN, BTILE), jnp.float32),
            pltpu.VMEM((SEQ_LEN // 2, 2 * HIDDEN), jnp.float32),
            pltpu.SemaphoreType.DMA,
            pltpu.SemaphoreType.DMA,
            pltpu.SemaphoreType.DMA,
            pltpu.SemaphoreType.DMA,
        ],
        compiler_params=pltpu.CompilerParams(
            use_tc_tiling_on_sc=True, needs_layout_passes=False),
    )(_sc_body)
    return fn(idx, table, pe)


def kernel(inputs, table):
    idx = inputs.T.astype(jnp.int32)
    table = jnp.pad(table.astype(jnp.float32), ((0, 0), (0, HPAD - HIDDEN)))
    pe = jnp.asarray(_PE, dtype=jnp.float32)
    out = _sc_embed(idx, table, pe)
    return jnp.transpose(out, (2, 0, 1))
